# Initial kernel scaffold; baseline (speedup 1.0000x reference)
import jax
import jax.numpy as jnp
from jax.experimental import pallas as pl


def kernel(x, adj, W, b):
    raise NotImplementedError("write your pallas kernel here")


# speedup vs baseline: 1.0592x; 1.0592x over previous
import functools

import jax
import jax.numpy as jnp
from jax.experimental import pallas as pl
from jax.experimental.pallas import tpu as pltpu

B, N, D = 2, 4096, 128
TI = 512


def _prep_body(adj_ref, x_ref, w_ref, s_ref, norm_ref):
    deg = jnp.sum(jnp.abs(adj_ref[0]), axis=-1)
    norm = jax.lax.rsqrt(deg + 1e-6)
    s = jnp.dot(x_ref[0], w_ref[...], preferred_element_type=jnp.float32)
    s_ref[0] = s * norm[:, None]
    norm_ref[0, 0] = norm


def _mm_body(adj_ref, s_ref, norm_ref, bias_ref, out_ref):
    acc = jnp.dot(adj_ref[0], s_ref[0], preferred_element_type=jnp.float32)
    out = acc * norm_ref[0, 0][:, None] + bias_ref[0]
    out_ref[0] = jnp.maximum(out, 0.0)


@functools.partial(jax.jit, static_argnames=())
def kernel(x, adj, W, b):
    grid = (B, N // TI)

    s_norm, norm = pl.pallas_call(
        _prep_body,
        grid=grid,
        in_specs=[
            pl.BlockSpec((1, TI, N), lambda bb, i: (bb, i, 0)),
            pl.BlockSpec((1, TI, D), lambda bb, i: (bb, i, 0)),
            pl.BlockSpec((D, D), lambda bb, i: (0, 0)),
        ],
        out_specs=[
            pl.BlockSpec((1, TI, D), lambda bb, i: (bb, i, 0)),
            pl.BlockSpec((1, 1, TI), lambda bb, i: (bb, 0, i)),
        ],
        out_shape=[
            jax.ShapeDtypeStruct((B, N, D), jnp.float32),
            jax.ShapeDtypeStruct((B, 1, N), jnp.float32),
        ],
        compiler_params=pltpu.CompilerParams(
            dimension_semantics=("parallel", "parallel"),
        ),
    )(adj, x, W)

    out = pl.pallas_call(
        _mm_body,
        grid=grid,
        in_specs=[
            pl.BlockSpec((1, TI, N), lambda bb, i: (bb, i, 0)),
            pl.BlockSpec((1, N, D), lambda bb, i: (bb, 0, 0)),
            pl.BlockSpec((1, 1, TI), lambda bb, i: (bb, 0, i)),
            pl.BlockSpec((1, D), lambda bb, i: (0, 0)),
        ],
        out_specs=pl.BlockSpec((1, TI, D), lambda bb, i: (bb, i, 0)),
        out_shape=jax.ShapeDtypeStruct((B, N, D), jnp.float32),
        compiler_params=pltpu.CompilerParams(
            dimension_semantics=("parallel", "parallel"),
        ),
    )(adj, s_norm, norm, b.reshape(1, D))

    return out


# speedup vs baseline: 1.4648x; 1.3830x over previous
import jax
import jax.numpy as jnp
from jax.experimental import pallas as pl
from jax.experimental.pallas import tpu as pltpu

B, N, D = 2, 4096, 128
NC = 32
TC_ = N // NC
RING = 4
TI = 512


def _body(adj_hbm, x_ref, w_ref, bias_ref, out_ref,
          ring_ref, adj_bf16, nrm_ref, s_ref, sem):
    bb = pl.program_id(0)

    def chunk_copy(c):
        return pltpu.make_async_copy(
            adj_hbm.at[bb, pl.ds(c * TC_, TC_), :],
            ring_ref.at[c % RING],
            sem.at[c % RING],
        )

    for c in range(RING):
        chunk_copy(c).start()

    def stream_step(c, _):
        chunk_copy(c).wait()
        slot = jax.lax.rem(c, RING)
        deg = jnp.sum(jnp.abs(ring_ref[slot]), axis=-1,
                      keepdims=True)
        nrm_ref[pl.ds(c * TC_, TC_), :] = jax.lax.rsqrt(deg + 1e-6)
        adj_bf16[pl.ds(c * TC_, TC_), :] = ring_ref[slot].astype(jnp.bfloat16)

        @pl.when(c + RING < NC)
        def _():
            chunk_copy(c + RING).start()

        return 0

    jax.lax.fori_loop(0, NC, stream_step, 0)

    s = jnp.dot(x_ref[0], w_ref[...], preferred_element_type=jnp.float32)
    s_ref[...] = (s * nrm_ref[...]).astype(jnp.bfloat16)

    for i in range(N // TI):
        acc = jnp.dot(adj_bf16[pl.ds(i * TI, TI), :], s_ref[...],
                      preferred_element_type=jnp.float32)
        out = acc * nrm_ref[pl.ds(i * TI, TI), :] + bias_ref[0]
        out_ref[0, pl.ds(i * TI, TI), :] = jnp.maximum(out, 0.0)


@jax.jit
def kernel(x, adj, W, b):
    return pl.pallas_call(
        _body,
        grid=(B,),
        in_specs=[
            pl.BlockSpec(memory_space=pl.ANY),
            pl.BlockSpec((1, N, D), lambda bb: (bb, 0, 0)),
            pl.BlockSpec((D, D), lambda bb: (0, 0)),
            pl.BlockSpec((1, D), lambda bb: (0, 0)),
        ],
        out_specs=pl.BlockSpec((1, N, D), lambda bb: (bb, 0, 0)),
        out_shape=jax.ShapeDtypeStruct((B, N, D), jnp.float32),
        scratch_shapes=[
            pltpu.VMEM((RING, TC_, N), jnp.float32),
            pltpu.VMEM((N, N), jnp.bfloat16),
            pltpu.VMEM((N, 1), jnp.float32),
            pltpu.VMEM((N, D), jnp.bfloat16),
            pltpu.SemaphoreType.DMA((RING,)),
        ],
        compiler_params=pltpu.CompilerParams(
            dimension_semantics=("arbitrary",),
        ),
    )(adj, x, W, b.reshape(1, D))


# speedup vs baseline: 1.6048x; 1.0956x over previous
import jax
import jax.numpy as jnp
from jax.experimental import pallas as pl
from jax.experimental.pallas import tpu as pltpu

B, N, D = 2, 4096, 128
NC = 32
TC_ = N // NC
RING = 4
TI = 512
NB = N // TI
CPB = TI // TC_


def _body(adj_hbm, x_ref, w_ref, bias_ref, out_ref,
          ring_ref, adj_bf16, nrm_ref, s_ref, sem):
    t = pl.program_id(0)
    streaming = t < B
    matmuling = t > 0
    bsrc = jnp.minimum(t, B - 1)
    cur = jax.lax.rem(t, 2)
    prv = jax.lax.rem(t + 1, 2)

    def chunk_copy(c):
        slot = jax.lax.rem(c, RING)
        return pltpu.make_async_copy(
            adj_hbm.at[bsrc, pl.ds(c * TC_, TC_), :],
            ring_ref.at[slot],
            sem.at[slot],
        )

    @pl.when(streaming)
    def _():
        for c in range(RING):
            chunk_copy(c).start()

    def block_step(i, _):
        @pl.when(matmuling)
        def _():
            acc = jnp.dot(adj_bf16[pl.ds(i * TI, TI), :], s_ref[...],
                          preferred_element_type=jnp.float32)
            out = acc * nrm_ref[prv, pl.ds(i * TI, TI), :] + bias_ref[0]
            out_ref[0, pl.ds(i * TI, TI), :] = jnp.maximum(out, 0.0)

        def chunk_step(k, _):
            c = i * CPB + k

            @pl.when(streaming)
            def _():
                chunk_copy(c).wait()
                slot = jax.lax.rem(c, RING)
                deg = jnp.sum(jnp.abs(ring_ref[slot]), axis=-1,
                              keepdims=True)
                nrm_ref[cur, pl.ds(c * TC_, TC_), :] = jax.lax.rsqrt(deg + 1e-6)
                adj_bf16[pl.ds(c * TC_, TC_), :] = (
                    ring_ref[slot].astype(jnp.bfloat16))

                @pl.when(c + RING < NC)
                def _():
                    chunk_copy(c + RING).start()

            return 0

        jax.lax.fori_loop(0, CPB, chunk_step, 0)
        return 0

    jax.lax.fori_loop(0, NB, block_step, 0)

    @pl.when(streaming)
    def _():
        s = jnp.dot(x_ref[0], w_ref[...], preferred_element_type=jnp.float32)
        s_ref[...] = (s * nrm_ref[cur]).astype(jnp.bfloat16)


@jax.jit
def kernel(x, adj, W, b):
    return pl.pallas_call(
        _body,
        grid=(B + 1,),
        in_specs=[
            pl.BlockSpec(memory_space=pl.ANY),
            pl.BlockSpec((1, N, D), lambda t: (jnp.minimum(t, B - 1), 0, 0)),
            pl.BlockSpec((D, D), lambda t: (0, 0)),
            pl.BlockSpec((1, D), lambda t: (0, 0)),
        ],
        out_specs=pl.BlockSpec(
            (1, N, D), lambda t: (jnp.maximum(t - 1, 0), 0, 0)),
        out_shape=jax.ShapeDtypeStruct((B, N, D), jnp.float32),
        scratch_shapes=[
            pltpu.VMEM((RING, TC_, N), jnp.float32),
            pltpu.VMEM((N, N), jnp.bfloat16),
            pltpu.VMEM((2, N, 1), jnp.float32),
            pltpu.VMEM((N, D), jnp.bfloat16),
            pltpu.SemaphoreType.DMA((RING,)),
        ],
        compiler_params=pltpu.CompilerParams(
            dimension_semantics=("arbitrary",),
        ),
    )(adj, x, W, b.reshape(1, D))
